# Initial kernel scaffold; baseline (speedup 1.0000x reference)
import jax
import jax.numpy as jnp
from jax.experimental import pallas as pl


def kernel(pos, batch, params):
    raise NotImplementedError("write your pallas kernel here")


# speedup vs baseline: 9.6592x; 9.6592x over previous
import functools

import numpy as np
import jax
import jax.numpy as jnp
from jax import lax
from jax.experimental import pallas as pl
from jax.experimental.pallas import tpu as pltpu, tpu_sc as plsc

BB = 16
NPTS = 1024
KNBR = 64
NEG = -1e30
_HI = jax.lax.Precision.DEFAULT


def _mm(a, b):
    return jnp.dot(a, b, precision=_HI)
_BN_SC = 1.0 / np.sqrt(1.0 + 1e-5)


def _fps_call(pos_t, n, n_sel):
    def body(pos_ref, out_ref):
        px = pos_ref[:, 0, :]
        py = pos_ref[:, 1, :]
        pz = pos_ref[:, 2, :]
        iota = lax.broadcasted_iota(jnp.int32, (BB, n), 1)

        def step(s, carry):
            dists, last = carry
            onehot = iota == last
            lx = jnp.sum(jnp.where(onehot, px, 0.0), axis=1, keepdims=True)
            ly = jnp.sum(jnp.where(onehot, py, 0.0), axis=1, keepdims=True)
            lz = jnp.sum(jnp.where(onehot, pz, 0.0), axis=1, keepdims=True)
            out_ref[s] = jnp.concatenate([lx, ly, lz], axis=1)
            dx = px - lx
            dy = py - ly
            dz = pz - lz
            d = dx * dx + dy * dy + dz * dz
            dists = jnp.minimum(dists, d)
            m = jnp.max(dists, axis=1, keepdims=True)
            nxt = jnp.min(jnp.where(dists == m, iota, n), axis=1, keepdims=True)
            return dists, nxt

        lax.fori_loop(
            0, n_sel, step,
            (jnp.full((BB, n), 1e30, jnp.float32), jnp.zeros((BB, 1), jnp.int32)),
        )

    return pl.pallas_call(
        body,
        out_shape=jax.ShapeDtypeStruct((n_sel, BB, 3), jnp.float32),
    )(pos_t)


def _group_call(pos_t, cen_t, n, m, radius, emit_nbr):
    info = plsc.get_sparse_core_info()
    nw = info.num_cores * info.num_subcores
    cpw = nw // BB
    mc = m // cpw
    ngrp = n // 16
    r2 = radius * radius

    out_type = [
        jax.ShapeDtypeStruct((BB, m * KNBR * 3), jnp.float32),
        jax.ShapeDtypeStruct((BB, m * KNBR), jnp.float32),
    ]
    scratch = [
        pltpu.VMEM((n,), jnp.float32), pltpu.VMEM((n,), jnp.float32),
        pltpu.VMEM((n,), jnp.float32),
        pltpu.VMEM((mc,), jnp.float32), pltpu.VMEM((mc,), jnp.float32),
        pltpu.VMEM((mc,), jnp.float32),
        pltpu.VMEM((mc * KNBR * 3,), jnp.float32),
        pltpu.VMEM((mc * KNBR,), jnp.float32),
    ]
    if emit_nbr:
        out_type.append(jax.ShapeDtypeStruct((BB, m * KNBR), jnp.int32))
        scratch.append(pltpu.VMEM((mc * KNBR,), jnp.int32))

    mesh = plsc.VectorSubcoreMesh(core_axis_name="c", subcore_axis_name="s")

    def body(*refs):
        if emit_nbr:
            (pxh, pyh, pzh, cxh, cyh, czh, relh, maskh, nbrh,
             px, py, pz, cx, cy, cz, relv, maskv, nbrv) = refs
        else:
            (pxh, pyh, pzh, cxh, cyh, czh, relh, maskh,
             px, py, pz, cx, cy, cz, relv, maskv) = refs
            nbrv = None
        wid = lax.axis_index("s") * info.num_cores + lax.axis_index("c")
        cloud = wid // cpw
        chunk = wid % cpw
        cstart = chunk * mc

        pltpu.sync_copy(pxh.at[cloud], px)
        pltpu.sync_copy(pyh.at[cloud], py)
        pltpu.sync_copy(pzh.at[cloud], pz)
        pltpu.sync_copy(cxh.at[cloud, pl.ds(cstart, mc)], cx)
        pltpu.sync_copy(cyh.at[cloud, pl.ds(cstart, mc)], cy)
        pltpu.sync_copy(czh.at[cloud, pl.ds(cstart, mc)], cz)

        zero16 = jnp.zeros((16,), jnp.float32)
        neg16 = jnp.full((16,), NEG, jnp.float32)
        izero16 = jnp.zeros((16,), jnp.int32)

        def fill(k, _):
            maskv[pl.ds(k * 16, 16)] = neg16
            if emit_nbr:
                nbrv[pl.ds(k * 16, 16)] = izero16
            return 0

        lax.fori_loop(0, mc * KNBR // 16, fill, 0)

        def fillr(k, _):
            relv[pl.ds(k * 16, 16)] = zero16
            return 0

        lax.fori_loop(0, mc * KNBR * 3 // 16, fillr, 0)

        lane = lax.iota(jnp.int32, 16)
        goff = cloud * n

        def per_center(i, _):
            ii = jnp.full((16,), i, jnp.int32)
            ccx = plsc.load_gather(cx, [ii])
            ccy = plsc.load_gather(cy, [ii])
            ccz = plsc.load_gather(cz, [ii])
            base = i * KNBR

            def per_group(g, cnt):
                for u in range(4):
                    off = (g * 4 + u) * 16
                    vx = px[pl.ds(off, 16)]
                    vy = py[pl.ds(off, 16)]
                    vz = pz[pl.ds(off, 16)]
                    dx = vx - ccx
                    dy = vy - ccy
                    dz = vz - ccz
                    d2 = dx * dx + dy * dy + dz * dz
                    ok = d2 <= r2
                    pref = plsc.cumsum(jnp.where(ok, 1, 0))
                    pos = cnt + pref - 1
                    okw = ok & (pos < KNBR)
                    slot = base + pos
                    plsc.store_scatter(relv, [slot * 3], dx, mask=okw)
                    plsc.store_scatter(relv, [slot * 3 + 1], dy, mask=okw)
                    plsc.store_scatter(relv, [slot * 3 + 2], dz, mask=okw)
                    plsc.store_scatter(maskv, [slot], zero16, mask=okw)
                    if emit_nbr:
                        plsc.store_scatter(nbrv, [slot], goff + off + lane,
                                           mask=okw)
                    cnt = cnt + plsc.all_reduce_population_count(ok)
                return cnt

            lax.fori_loop(0, ngrp // 4, per_group, jnp.zeros((16,), jnp.int32))
            return 0

        lax.fori_loop(0, mc, per_center, 0)

        obase = chunk * (mc * KNBR)
        pltpu.sync_copy(relv, relh.at[cloud, pl.ds(obase * 3, mc * KNBR * 3)])
        pltpu.sync_copy(maskv, maskh.at[cloud, pl.ds(obase, mc * KNBR)])
        if emit_nbr:
            pltpu.sync_copy(nbrv, nbrh.at[cloud, pl.ds(obase, mc * KNBR)])

    fn = pl.kernel(body, out_type=tuple(out_type), mesh=mesh,
                   scratch_types=scratch,
                   compiler_params=pltpu.CompilerParams(
                       needs_layout_passes=False))
    return fn(pos_t[:, 0, :], pos_t[:, 1, :], pos_t[:, 2, :],
              cen_t[:, 0, :], cen_t[:, 1, :], cen_t[:, 2, :])


def _gather_rows(table, idx):
    q, d = idx.shape[0], table.shape[1]
    info = plsc.get_sparse_core_info()
    nw = info.num_cores * info.num_subcores
    per_w = q // nw
    chunk = 128
    nch = per_w // chunk
    mesh = plsc.VectorSubcoreMesh(core_axis_name="c", subcore_axis_name="s")

    def body(tab_h, idx_h, out_h, idx_v, rows_v, sem):
        wid = lax.axis_index("s") * info.num_cores + lax.axis_index("c")

        def step(t, _):
            base = wid * per_w + t * chunk
            pltpu.sync_copy(idx_h.at[pl.ds(base, chunk)], idx_v)
            pltpu.async_copy(tab_h.at[idx_v], rows_v, sem).wait()
            pltpu.sync_copy(rows_v, out_h.at[pl.ds(base, chunk)])
            return 0

        lax.fori_loop(0, nch, step, 0)

    fn = pl.kernel(
        body,
        out_type=jax.ShapeDtypeStruct((q, d), jnp.float32),
        mesh=mesh,
        scratch_types=[
            pltpu.VMEM((chunk,), jnp.int32),
            pltpu.VMEM((chunk, d), jnp.float32),
            pltpu.SemaphoreType.DMA,
        ],
        compiler_params=pltpu.CompilerParams(needs_layout_passes=False),
    )
    return fn(table, idx)


def _mlp_pool_call(xg, rel, mask, ws, n_centers, cblk):
    rows_blk = cblk * KNBR
    grid = n_centers // cblk
    (w0, b0, g0, be0), (w1, b1, g1, be1), (w2, b2) = ws

    def body(*refs):
        if xg is None:
            (rel_ref, mask_ref, w0_r, b0_r, g0_r, be0_r, w1_r, b1_r, g1_r,
             be1_r, w2_r, b2_r, out_ref) = refs
            x = rel_ref[...]
        else:
            (xg_ref, rel_ref, mask_ref, w0_r, b0_r, g0_r, be0_r, w1_r, b1_r,
             g1_r, be1_r, w2_r, b2_r, out_ref) = refs
            x = jnp.concatenate([xg_ref[...], rel_ref[...]], axis=1)
        h = _mm(x, w0_r[...]) + b0_r[...]
        h = jnp.maximum(g0_r[...] * (h * _BN_SC) + be0_r[...], 0.0)
        h = _mm(h, w1_r[...]) + b1_r[...]
        h = jnp.maximum(g1_r[...] * (h * _BN_SC) + be1_r[...], 0.0)
        h = _mm(h, w2_r[...]) + b2_r[...]
        c_out = h.shape[-1]
        h = h.reshape(cblk, KNBR, c_out) + mask_ref[...][:, :, None]
        out_ref[...] = jnp.max(h, axis=1)

    full = lambda a: pl.BlockSpec(a.shape, lambda i: (0,) * a.ndim)
    in_specs = []
    args = []
    if xg is not None:
        in_specs.append(pl.BlockSpec((rows_blk, xg.shape[1]), lambda i: (i, 0)))
        args.append(xg)
    in_specs.append(pl.BlockSpec((rows_blk, 3), lambda i: (i, 0)))
    args.append(rel)
    in_specs.append(pl.BlockSpec((cblk, KNBR), lambda i: (i, 0)))
    args.append(mask)
    wlist = [w0, b0, g0, be0, w1, b1, g1, be1, w2, b2]
    for w in wlist:
        in_specs.append(full(w))
        args.append(w)
    c_out = w2.shape[1]
    return pl.pallas_call(
        body,
        grid=(grid,),
        in_specs=in_specs,
        out_specs=pl.BlockSpec((cblk, c_out), lambda i: (i, 0)),
        out_shape=jax.ShapeDtypeStruct((n_centers, c_out), jnp.float32),
    )(*args)


def _head_call(x2, p2, w3s, wpi, wval):
    npts = x2.shape[0] // BB
    (w0, b0, g0, be0), (w1, b1, g1, be1), (w2, b2) = w3s
    (p0, pb0), (p1, pb1), (p2w, pb2) = wpi
    (v0, vb0), (v1, vb1), (v2, vb2) = wval

    def body(x2_ref, p2_ref, w0_r, b0_r, g0_r, be0_r, w1_r, b1_r, g1_r,
             be1_r, w2_r, b2_r,
             p0_r, pb0_r, p1_r, pb1_r, p2_r, pb2_r,
             v0_r, vb0_r, v1_r, vb1_r, v2_r, vb2_r,
             probs_ref, val_ref):
        x = jnp.concatenate([x2_ref[...], p2_ref[...]], axis=1)
        h = _mm(x, w0_r[...]) + b0_r[...]
        h = jnp.maximum(g0_r[...] * (h * _BN_SC) + be0_r[...], 0.0)
        h = _mm(h, w1_r[...]) + b1_r[...]
        h = jnp.maximum(g1_r[...] * (h * _BN_SC) + be1_r[...], 0.0)
        h = _mm(h, w2_r[...]) + b2_r[...]
        feats = jnp.max(h.reshape(BB, npts, h.shape[-1]), axis=1)
        g = _mm(feats, p0_r[...]) + pb0_r[...]
        g = _mm(g, p1_r[...]) + pb1_r[...]
        logits = _mm(g, p2_r[...]) + pb2_r[...]
        mlog = jnp.max(logits, axis=1, keepdims=True)
        e = jnp.exp(logits - mlog)
        probs_ref[...] = e / jnp.sum(e, axis=1, keepdims=True)
        v = _mm(feats, v0_r[...]) + vb0_r[...]
        v = _mm(v, v1_r[...]) + vb1_r[...]
        val_ref[...] = _mm(v, v2_r[...]) + vb2_r[...]

    args = [x2, p2, w0, b0, g0, be0, w1, b1, g1, be1, w2, b2,
            p0, pb0, p1, pb1, p2w, pb2, v0, vb0, v1, vb1, v2, vb2]
    return pl.pallas_call(
        body,
        out_shape=(jax.ShapeDtypeStruct((BB, 10), jnp.float32),
                   jax.ShapeDtypeStruct((BB, 1), jnp.float32)),
    )(*args)


def _bn_params(p):
    out = []
    for i in range(3):
        w, b = p["Ws"][i], p["bs"][i]
        if i < 2:
            out.append((w, b[None, :], p["gammas"][i][None, :],
                        p["betas"][i][None, :]))
        else:
            out.append((w, b[None, :]))
    return out


@jax.jit
def kernel(pos, batch, params):
    del batch
    pos_t = pos.reshape(BB, NPTS, 3).transpose(0, 2, 1)

    c1 = _fps_call(pos_t, NPTS, NPTS // 2)
    c1_t = c1.transpose(1, 2, 0)
    rel1, mask1 = _group_call(pos_t, c1_t, NPTS, 512, 0.2, False)

    c2 = _fps_call(c1_t, 512, 128)
    c2_t = c2.transpose(1, 2, 0)
    rel2, mask2, nbr2 = _group_call(c1_t, c2_t, 512, 128, 0.4, True)

    x1 = _mlp_pool_call(
        None, rel1.reshape(BB * 512 * KNBR, 3),
        mask1.reshape(BB * 512, KNBR),
        _bn_params(params["sa1"]), BB * 512, 128)

    xg2 = _gather_rows(x1, nbr2.reshape(-1))

    x2 = _mlp_pool_call(
        xg2, rel2.reshape(BB * 128 * KNBR, 3),
        mask2.reshape(BB * 128, KNBR),
        _bn_params(params["sa2"]), BB * 128, 64)

    ws3 = _bn_params(params["sa3"])
    wpi = [(w, b[None, :]) for w, b in
           zip(params["pi"]["Ws"], params["pi"]["bs"])]
    wval = [(w, b[None, :]) for w, b in
            zip(params["value"]["Ws"], params["value"]["bs"])]
    p2f = c2.transpose(1, 0, 2).reshape(BB * 128, 3)
    probs, value = _head_call(x2, p2f, ws3, wpi, wval)
    return probs, value[:, 0]


# speedup vs baseline: 10.1620x; 1.0521x over previous
import functools

import numpy as np
import jax
import jax.numpy as jnp
from jax import lax
from jax.experimental import pallas as pl
from jax.experimental.pallas import tpu as pltpu, tpu_sc as plsc

BB = 16
NPTS = 1024
KNBR = 64
NEG = -1e30
_HI = jax.lax.Precision.DEFAULT


def _mm(a, b):
    return jnp.dot(a, b, precision=_HI)
_BN_SC = 1.0 / np.sqrt(1.0 + 1e-5)


def _fps_call(pos_t, n, n_sel):
    def body(pos_ref, out_ref):
        px = pos_ref[:, 0, :]
        py = pos_ref[:, 1, :]
        pz = pos_ref[:, 2, :]
        iota = lax.broadcasted_iota(jnp.int32, (BB, n), 1)

        def step(s, carry):
            dists, last = carry
            onehot = iota == last
            lx = jnp.sum(jnp.where(onehot, px, 0.0), axis=1, keepdims=True)
            ly = jnp.sum(jnp.where(onehot, py, 0.0), axis=1, keepdims=True)
            lz = jnp.sum(jnp.where(onehot, pz, 0.0), axis=1, keepdims=True)
            out_ref[s] = jnp.concatenate([lx, ly, lz], axis=1)
            dx = px - lx
            dy = py - ly
            dz = pz - lz
            d = dx * dx + dy * dy + dz * dz
            dists = jnp.minimum(dists, d)
            m = jnp.max(dists, axis=1, keepdims=True)
            nxt = jnp.min(jnp.where(dists == m, iota, n), axis=1, keepdims=True)
            return dists, nxt

        lax.fori_loop(
            0, n_sel, step,
            (jnp.full((BB, n), 1e30, jnp.float32), jnp.zeros((BB, 1), jnp.int32)),
        )

    return pl.pallas_call(
        body,
        out_shape=jax.ShapeDtypeStruct((n_sel, BB, 3), jnp.float32),
    )(pos_t)


def _group_call(pos_t, cen_t, n, m, radius, emit_nbr):
    info = plsc.get_sparse_core_info()
    nw = info.num_cores * info.num_subcores
    cpw = nw // BB
    mc = m // cpw
    ngrp = n // 16
    r2 = radius * radius

    out_type = [
        jax.ShapeDtypeStruct((BB, m * KNBR * 3), jnp.float32),
        jax.ShapeDtypeStruct((BB, m * KNBR), jnp.float32),
    ]
    scratch = [
        pltpu.VMEM((n,), jnp.float32), pltpu.VMEM((n,), jnp.float32),
        pltpu.VMEM((n,), jnp.float32),
        pltpu.VMEM((mc,), jnp.float32), pltpu.VMEM((mc,), jnp.float32),
        pltpu.VMEM((mc,), jnp.float32),
        pltpu.VMEM((mc * KNBR * 3,), jnp.float32),
        pltpu.VMEM((mc * KNBR,), jnp.float32),
    ]
    if emit_nbr:
        out_type.append(jax.ShapeDtypeStruct((BB, m * KNBR), jnp.int32))
        scratch.append(pltpu.VMEM((mc * KNBR,), jnp.int32))

    mesh = plsc.VectorSubcoreMesh(core_axis_name="c", subcore_axis_name="s")

    def body(*refs):
        if emit_nbr:
            (pxh, pyh, pzh, cxh, cyh, czh, relh, maskh, nbrh,
             px, py, pz, cx, cy, cz, relv, maskv, nbrv) = refs
        else:
            (pxh, pyh, pzh, cxh, cyh, czh, relh, maskh,
             px, py, pz, cx, cy, cz, relv, maskv) = refs
            nbrv = None
        wid = lax.axis_index("s") * info.num_cores + lax.axis_index("c")
        cloud = wid // cpw
        chunk = wid % cpw
        cstart = chunk * mc

        pltpu.sync_copy(pxh.at[cloud], px)
        pltpu.sync_copy(pyh.at[cloud], py)
        pltpu.sync_copy(pzh.at[cloud], pz)
        pltpu.sync_copy(cxh.at[cloud, pl.ds(cstart, mc)], cx)
        pltpu.sync_copy(cyh.at[cloud, pl.ds(cstart, mc)], cy)
        pltpu.sync_copy(czh.at[cloud, pl.ds(cstart, mc)], cz)

        zero16 = jnp.zeros((16,), jnp.float32)
        neg16 = jnp.full((16,), NEG, jnp.float32)
        izero16 = jnp.zeros((16,), jnp.int32)

        def fill(k, _):
            maskv[pl.ds(k * 16, 16)] = neg16
            if emit_nbr:
                nbrv[pl.ds(k * 16, 16)] = izero16
            return 0

        lax.fori_loop(0, mc * KNBR // 16, fill, 0)

        def fillr(k, _):
            relv[pl.ds(k * 16, 16)] = zero16
            return 0

        lax.fori_loop(0, mc * KNBR * 3 // 16, fillr, 0)

        lane = lax.iota(jnp.int32, 16)
        goff = cloud * n
        U = 4

        def per_block(blk, _):
            ccx = cx[pl.ds(blk * 16, 16)]
            ccy = cy[pl.ds(blk * 16, 16)]
            ccz = cz[pl.ds(blk * 16, 16)]
            base = (blk * 16 + lane) * KNBR

            def per_pt(jj, cnt):
                for u in range(U):
                    j = jj * U + u
                    jv = jnp.full((16,), j, jnp.int32)
                    vx = plsc.load_gather(px, [jv])
                    vy = plsc.load_gather(py, [jv])
                    vz = plsc.load_gather(pz, [jv])
                    dx = vx - ccx
                    dy = vy - ccy
                    dz = vz - ccz
                    d2 = dx * dx + dy * dy + dz * dz
                    okw = (d2 <= r2) & (cnt < KNBR)
                    slot = base + cnt
                    s3 = slot * 3
                    plsc.store_scatter(relv, [s3], dx, mask=okw)
                    plsc.store_scatter(relv, [s3 + 1], dy, mask=okw)
                    plsc.store_scatter(relv, [s3 + 2], dz, mask=okw)
                    plsc.store_scatter(maskv, [slot], zero16, mask=okw)
                    if emit_nbr:
                        plsc.store_scatter(nbrv, [slot], goff + jv, mask=okw)
                    cnt = cnt + jnp.where(okw, 1, 0)
                return cnt

            lax.fori_loop(0, n // U, per_pt, jnp.zeros((16,), jnp.int32))
            return 0

        lax.fori_loop(0, mc // 16, per_block, 0)

        obase = chunk * (mc * KNBR)
        pltpu.sync_copy(relv, relh.at[cloud, pl.ds(obase * 3, mc * KNBR * 3)])
        pltpu.sync_copy(maskv, maskh.at[cloud, pl.ds(obase, mc * KNBR)])
        if emit_nbr:
            pltpu.sync_copy(nbrv, nbrh.at[cloud, pl.ds(obase, mc * KNBR)])

    fn = pl.kernel(body, out_type=tuple(out_type), mesh=mesh,
                   scratch_types=scratch,
                   compiler_params=pltpu.CompilerParams(
                       needs_layout_passes=False))
    return fn(pos_t[:, 0, :], pos_t[:, 1, :], pos_t[:, 2, :],
              cen_t[:, 0, :], cen_t[:, 1, :], cen_t[:, 2, :])


def _gather_rows(table, idx):
    q, d = idx.shape[0], table.shape[1]
    info = plsc.get_sparse_core_info()
    nw = info.num_cores * info.num_subcores
    per_w = q // nw
    chunk = 128
    nch = per_w // chunk
    mesh = plsc.VectorSubcoreMesh(core_axis_name="c", subcore_axis_name="s")

    nb = 3

    def body(tab_h, idx_h, out_h, idx_v, rows_v, sem_g, sem_o):
        wid = lax.axis_index("s") * info.num_cores + lax.axis_index("c")
        base = wid * per_w
        pltpu.sync_copy(idx_h.at[pl.ds(base, per_w)], idx_v)

        def gat(t):
            return pltpu.async_copy(
                tab_h.at[idx_v.at[pl.ds(t * chunk, chunk)]],
                rows_v.at[t % nb], sem_g.at[t % nb])

        def put(t):
            return pltpu.async_copy(
                rows_v.at[t % nb],
                out_h.at[pl.ds(base + t * chunk, chunk)], sem_o.at[t % nb])

        gh = {0: gat(0), 1: gat(1)}
        wh = {}
        for t in range(nch):
            gh[t].wait()
            wh[t] = put(t)
            nt = t + 2
            if nt < nch:
                if nt >= nb:
                    wh[nt - nb].wait()
                gh[nt] = gat(nt)
        wh[nch - 3].wait()
        wh[nch - 2].wait()
        wh[nch - 1].wait()

    fn = pl.kernel(
        body,
        out_type=jax.ShapeDtypeStruct((q, d), jnp.float32),
        mesh=mesh,
        scratch_types=[
            pltpu.VMEM((per_w,), jnp.int32),
            pltpu.VMEM((nb, chunk, d), jnp.float32),
            pltpu.SemaphoreType.DMA((nb,)),
            pltpu.SemaphoreType.DMA((nb,)),
        ],
        compiler_params=pltpu.CompilerParams(needs_layout_passes=False),
    )
    return fn(table, idx)


def _mlp_pool_call(xg, rel, mask, ws, n_centers, cblk):
    rows_blk = cblk * KNBR
    grid = n_centers // cblk
    (w0, b0, g0, be0), (w1, b1, g1, be1), (w2, b2) = ws

    def body(*refs):
        if xg is None:
            (rel_ref, mask_ref, w0_r, b0_r, g0_r, be0_r, w1_r, b1_r, g1_r,
             be1_r, w2_r, b2_r, out_ref) = refs
            x = rel_ref[...]
        else:
            (xg_ref, rel_ref, mask_ref, w0_r, b0_r, g0_r, be0_r, w1_r, b1_r,
             g1_r, be1_r, w2_r, b2_r, out_ref) = refs
            x = jnp.concatenate([xg_ref[...], rel_ref[...]], axis=1)
        h = _mm(x, w0_r[...]) + b0_r[...]
        h = jnp.maximum(g0_r[...] * (h * _BN_SC) + be0_r[...], 0.0)
        h = _mm(h, w1_r[...]) + b1_r[...]
        h = jnp.maximum(g1_r[...] * (h * _BN_SC) + be1_r[...], 0.0)
        h = _mm(h, w2_r[...]) + b2_r[...]
        c_out = h.shape[-1]
        h = h.reshape(cblk, KNBR, c_out) + mask_ref[...][:, :, None]
        out_ref[...] = jnp.max(h, axis=1)

    full = lambda a: pl.BlockSpec(a.shape, lambda i: (0,) * a.ndim)
    in_specs = []
    args = []
    if xg is not None:
        in_specs.append(pl.BlockSpec((rows_blk, xg.shape[1]), lambda i: (i, 0)))
        args.append(xg)
    in_specs.append(pl.BlockSpec((rows_blk, 3), lambda i: (i, 0)))
    args.append(rel)
    in_specs.append(pl.BlockSpec((cblk, KNBR), lambda i: (i, 0)))
    args.append(mask)
    wlist = [w0, b0, g0, be0, w1, b1, g1, be1, w2, b2]
    for w in wlist:
        in_specs.append(full(w))
        args.append(w)
    c_out = w2.shape[1]
    return pl.pallas_call(
        body,
        grid=(grid,),
        in_specs=in_specs,
        out_specs=pl.BlockSpec((cblk, c_out), lambda i: (i, 0)),
        out_shape=jax.ShapeDtypeStruct((n_centers, c_out), jnp.float32),
    )(*args)


def _head_call(x2, p2, w3s, wpi, wval):
    npts = x2.shape[0] // BB
    (w0, b0, g0, be0), (w1, b1, g1, be1), (w2, b2) = w3s
    (p0, pb0), (p1, pb1), (p2w, pb2) = wpi
    (v0, vb0), (v1, vb1), (v2, vb2) = wval

    def body(x2_ref, p2_ref, w0_r, b0_r, g0_r, be0_r, w1_r, b1_r, g1_r,
             be1_r, w2_r, b2_r,
             p0_r, pb0_r, p1_r, pb1_r, p2_r, pb2_r,
             v0_r, vb0_r, v1_r, vb1_r, v2_r, vb2_r,
             probs_ref, val_ref):
        x = jnp.concatenate([x2_ref[...], p2_ref[...]], axis=1)
        h = _mm(x, w0_r[...]) + b0_r[...]
        h = jnp.maximum(g0_r[...] * (h * _BN_SC) + be0_r[...], 0.0)
        h = _mm(h, w1_r[...]) + b1_r[...]
        h = jnp.maximum(g1_r[...] * (h * _BN_SC) + be1_r[...], 0.0)
        h = _mm(h, w2_r[...]) + b2_r[...]
        feats = jnp.max(h.reshape(BB, npts, h.shape[-1]), axis=1)
        g = _mm(feats, p0_r[...]) + pb0_r[...]
        g = _mm(g, p1_r[...]) + pb1_r[...]
        logits = _mm(g, p2_r[...]) + pb2_r[...]
        mlog = jnp.max(logits, axis=1, keepdims=True)
        e = jnp.exp(logits - mlog)
        probs_ref[...] = e / jnp.sum(e, axis=1, keepdims=True)
        v = _mm(feats, v0_r[...]) + vb0_r[...]
        v = _mm(v, v1_r[...]) + vb1_r[...]
        val_ref[...] = _mm(v, v2_r[...]) + vb2_r[...]

    args = [x2, p2, w0, b0, g0, be0, w1, b1, g1, be1, w2, b2,
            p0, pb0, p1, pb1, p2w, pb2, v0, vb0, v1, vb1, v2, vb2]
    return pl.pallas_call(
        body,
        out_shape=(jax.ShapeDtypeStruct((BB, 10), jnp.float32),
                   jax.ShapeDtypeStruct((BB, 1), jnp.float32)),
    )(*args)


def _bn_params(p):
    out = []
    for i in range(3):
        w, b = p["Ws"][i], p["bs"][i]
        if i < 2:
            out.append((w, b[None, :], p["gammas"][i][None, :],
                        p["betas"][i][None, :]))
        else:
            out.append((w, b[None, :]))
    return out


@jax.jit
def kernel(pos, batch, params):
    del batch
    pos_t = pos.reshape(BB, NPTS, 3).transpose(0, 2, 1)

    c1 = _fps_call(pos_t, NPTS, NPTS // 2)
    c1_t = c1.transpose(1, 2, 0)
    rel1, mask1 = _group_call(pos_t, c1_t, NPTS, 512, 0.2, False)

    c2 = _fps_call(c1_t, 512, 128)
    c2_t = c2.transpose(1, 2, 0)
    rel2, mask2, nbr2 = _group_call(c1_t, c2_t, 512, 128, 0.4, True)

    x1 = _mlp_pool_call(
        None, rel1.reshape(BB * 512 * KNBR, 3),
        mask1.reshape(BB * 512, KNBR),
        _bn_params(params["sa1"]), BB * 512, 128)

    xg2 = _gather_rows(x1, nbr2.reshape(-1))

    x2 = _mlp_pool_call(
        xg2, rel2.reshape(BB * 128 * KNBR, 3),
        mask2.reshape(BB * 128, KNBR),
        _bn_params(params["sa2"]), BB * 128, 64)

    ws3 = _bn_params(params["sa3"])
    wpi = [(w, b[None, :]) for w, b in
           zip(params["pi"]["Ws"], params["pi"]["bs"])]
    wval = [(w, b[None, :]) for w, b in
            zip(params["value"]["Ws"], params["value"]["bs"])]
    p2f = c2.transpose(1, 0, 2).reshape(BB * 128, 3)
    probs, value = _head_call(x2, p2f, ws3, wpi, wval)
    return probs, value[:, 0]


# speedup vs baseline: 14.0786x; 1.3854x over previous
import functools

import numpy as np
import jax
import jax.numpy as jnp
from jax import lax
from jax.experimental import pallas as pl
from jax.experimental.pallas import tpu as pltpu, tpu_sc as plsc

BB = 16
NPTS = 1024
KNBR = 64
NEG = -1e30
_HI = jax.lax.Precision.DEFAULT


def _mm(a, b):
    return jnp.dot(a, b, precision=_HI)
_BN_SC = 1.0 / np.sqrt(1.0 + 1e-5)


def _fps_call(pos_t, n, n_sel):
    def body(pos_ref, out_ref):
        px = pos_ref[:, 0, :]
        py = pos_ref[:, 1, :]
        pz = pos_ref[:, 2, :]
        iota = lax.broadcasted_iota(jnp.int32, (BB, n), 1)

        def step(s, carry):
            dists, last = carry
            onehot = iota == last
            lx = jnp.sum(jnp.where(onehot, px, 0.0), axis=1, keepdims=True)
            ly = jnp.sum(jnp.where(onehot, py, 0.0), axis=1, keepdims=True)
            lz = jnp.sum(jnp.where(onehot, pz, 0.0), axis=1, keepdims=True)
            out_ref[s] = jnp.concatenate([lx, ly, lz], axis=1)
            dx = px - lx
            dy = py - ly
            dz = pz - lz
            d = dx * dx + dy * dy + dz * dz
            dists = jnp.minimum(dists, d)
            m = jnp.max(dists, axis=1, keepdims=True)
            nxt = jnp.min(jnp.where(dists == m, iota, n), axis=1, keepdims=True)
            return dists, nxt

        lax.fori_loop(
            0, n_sel, step,
            (jnp.full((BB, n), 1e30, jnp.float32), jnp.zeros((BB, 1), jnp.int32)),
        )

    return pl.pallas_call(
        body,
        out_shape=jax.ShapeDtypeStruct((n_sel, BB, 3), jnp.float32),
    )(pos_t)


def _group_call(pos_t, cen_t, n, m, radius, emit_nbr):
    info = plsc.get_sparse_core_info()
    nw = info.num_cores * info.num_subcores
    cpw = nw // BB
    mc = m // cpw
    ngrp = n // 16
    r2 = radius * radius

    out_type = [
        jax.ShapeDtypeStruct((BB, m * KNBR * 3), jnp.float32),
        jax.ShapeDtypeStruct((BB, m * KNBR), jnp.float32),
    ]
    scratch = [
        pltpu.VMEM((n,), jnp.float32), pltpu.VMEM((n,), jnp.float32),
        pltpu.VMEM((n,), jnp.float32),
        pltpu.VMEM((mc,), jnp.float32), pltpu.VMEM((mc,), jnp.float32),
        pltpu.VMEM((mc,), jnp.float32),
        pltpu.VMEM((mc * KNBR * 3,), jnp.float32),
        pltpu.VMEM((mc * KNBR,), jnp.float32),
    ]
    if emit_nbr:
        out_type.append(jax.ShapeDtypeStruct((BB, m * KNBR), jnp.int32))
        scratch.append(pltpu.VMEM((mc * KNBR,), jnp.int32))

    mesh = plsc.VectorSubcoreMesh(core_axis_name="c", subcore_axis_name="s")

    def body(*refs):
        if emit_nbr:
            (pxh, pyh, pzh, cxh, cyh, czh, relh, maskh, nbrh,
             px, py, pz, cx, cy, cz, relv, maskv, nbrv) = refs
        else:
            (pxh, pyh, pzh, cxh, cyh, czh, relh, maskh,
             px, py, pz, cx, cy, cz, relv, maskv) = refs
            nbrv = None
        wid = lax.axis_index("s") * info.num_cores + lax.axis_index("c")
        cloud = wid // cpw
        chunk = wid % cpw
        cstart = chunk * mc

        pltpu.sync_copy(pxh.at[cloud], px)
        pltpu.sync_copy(pyh.at[cloud], py)
        pltpu.sync_copy(pzh.at[cloud], pz)
        pltpu.sync_copy(cxh.at[cloud, pl.ds(cstart, mc)], cx)
        pltpu.sync_copy(cyh.at[cloud, pl.ds(cstart, mc)], cy)
        pltpu.sync_copy(czh.at[cloud, pl.ds(cstart, mc)], cz)

        zero16 = jnp.zeros((16,), jnp.float32)
        neg16 = jnp.full((16,), NEG, jnp.float32)
        izero16 = jnp.zeros((16,), jnp.int32)

        def fill(k, _):
            maskv[pl.ds(k * 16, 16)] = neg16
            if emit_nbr:
                nbrv[pl.ds(k * 16, 16)] = izero16
            return 0

        lax.fori_loop(0, mc * KNBR // 16, fill, 0)

        def fillr(k, _):
            relv[pl.ds(k * 16, 16)] = zero16
            return 0

        lax.fori_loop(0, mc * KNBR * 3 // 16, fillr, 0)

        lane = lax.iota(jnp.int32, 16)
        U = 4

        def per_block(blk, _):
            ccx = cx[pl.ds(blk * 16, 16)]
            ccy = cy[pl.ds(blk * 16, 16)]
            ccz = cz[pl.ds(blk * 16, 16)]
            base = (blk * 16 + lane) * KNBR

            def per_pt(jj, cnt):
                for u in range(U):
                    j = jj * U + u
                    jv = jnp.full((16,), j, jnp.int32)
                    vx = plsc.load_gather(px, [jv])
                    vy = plsc.load_gather(py, [jv])
                    vz = plsc.load_gather(pz, [jv])
                    dx = vx - ccx
                    dy = vy - ccy
                    dz = vz - ccz
                    d2 = dx * dx + dy * dy + dz * dz
                    okw = (d2 <= r2) & (cnt < KNBR)
                    slot = base + cnt
                    s3 = slot * 3
                    plsc.store_scatter(relv, [s3], dx, mask=okw)
                    plsc.store_scatter(relv, [s3 + 1], dy, mask=okw)
                    plsc.store_scatter(relv, [s3 + 2], dz, mask=okw)
                    plsc.store_scatter(maskv, [slot], zero16, mask=okw)
                    if emit_nbr:
                        plsc.store_scatter(nbrv, [slot], jv, mask=okw)
                    cnt = cnt + jnp.where(okw, 1, 0)
                return cnt

            lax.fori_loop(0, n // U, per_pt, jnp.zeros((16,), jnp.int32))
            return 0

        lax.fori_loop(0, mc // 16, per_block, 0)

        obase = chunk * (mc * KNBR)
        pltpu.sync_copy(relv, relh.at[cloud, pl.ds(obase * 3, mc * KNBR * 3)])
        pltpu.sync_copy(maskv, maskh.at[cloud, pl.ds(obase, mc * KNBR)])
        if emit_nbr:
            pltpu.sync_copy(nbrv, nbrh.at[cloud, pl.ds(obase, mc * KNBR)])

    fn = pl.kernel(body, out_type=tuple(out_type), mesh=mesh,
                   scratch_types=scratch,
                   compiler_params=pltpu.CompilerParams(
                       needs_layout_passes=False))
    return fn(pos_t[:, 0, :], pos_t[:, 1, :], pos_t[:, 2, :],
              cen_t[:, 0, :], cen_t[:, 1, :], cen_t[:, 2, :])


def _gather_rows(table, idx):
    q, d = idx.shape[0], table.shape[1]
    info = plsc.get_sparse_core_info()
    nw = info.num_cores * info.num_subcores
    per_w = q // nw
    chunk = 128
    nch = per_w // chunk
    mesh = plsc.VectorSubcoreMesh(core_axis_name="c", subcore_axis_name="s")

    nb = 3

    def body(tab_h, idx_h, out_h, idx_v, rows_v, sem_g, sem_o):
        wid = lax.axis_index("s") * info.num_cores + lax.axis_index("c")
        base = wid * per_w
        pltpu.sync_copy(idx_h.at[pl.ds(base, per_w)], idx_v)

        def gat(t):
            return pltpu.async_copy(
                tab_h.at[idx_v.at[pl.ds(t * chunk, chunk)]],
                rows_v.at[t % nb], sem_g.at[t % nb])

        def put(t):
            return pltpu.async_copy(
                rows_v.at[t % nb],
                out_h.at[pl.ds(base + t * chunk, chunk)], sem_o.at[t % nb])

        gh = {0: gat(0), 1: gat(1)}
        wh = {}
        for t in range(nch):
            gh[t].wait()
            wh[t] = put(t)
            nt = t + 2
            if nt < nch:
                if nt >= nb:
                    wh[nt - nb].wait()
                gh[nt] = gat(nt)
        wh[nch - 3].wait()
        wh[nch - 2].wait()
        wh[nch - 1].wait()

    fn = pl.kernel(
        body,
        out_type=jax.ShapeDtypeStruct((q, d), jnp.float32),
        mesh=mesh,
        scratch_types=[
            pltpu.VMEM((per_w,), jnp.int32),
            pltpu.VMEM((nb, chunk, d), jnp.float32),
            pltpu.SemaphoreType.DMA((nb,)),
            pltpu.SemaphoreType.DMA((nb,)),
        ],
        compiler_params=pltpu.CompilerParams(needs_layout_passes=False),
    )
    return fn(table, idx)


def _mlp_pool_call(xg, rel, mask, ws, n_centers, cblk):
    rows_blk = cblk * KNBR
    grid = n_centers // cblk
    (w0, b0, g0, be0), (w1, b1, g1, be1), (w2, b2) = ws

    def body(*refs):
        if xg is None:
            (rel_ref, mask_ref, w0_r, b0_r, g0_r, be0_r, w1_r, b1_r, g1_r,
             be1_r, w2_r, b2_r, out_ref) = refs
            x = rel_ref[...]
        else:
            (xg_ref, rel_ref, mask_ref, w0_r, b0_r, g0_r, be0_r, w1_r, b1_r,
             g1_r, be1_r, w2_r, b2_r, out_ref) = refs
            x = jnp.concatenate([xg_ref[...], rel_ref[...]], axis=1)
        h = _mm(x, w0_r[...]) + b0_r[...]
        h = jnp.maximum(g0_r[...] * (h * _BN_SC) + be0_r[...], 0.0)
        h = _mm(h, w1_r[...]) + b1_r[...]
        h = jnp.maximum(g1_r[...] * (h * _BN_SC) + be1_r[...], 0.0)
        h = _mm(h, w2_r[...]) + b2_r[...]
        c_out = h.shape[-1]
        h = h.reshape(cblk, KNBR, c_out) + mask_ref[...][:, :, None]
        out_ref[...] = jnp.max(h, axis=1)

    full = lambda a: pl.BlockSpec(a.shape, lambda i: (0,) * a.ndim)
    in_specs = []
    args = []
    if xg is not None:
        in_specs.append(pl.BlockSpec((rows_blk, xg.shape[1]), lambda i: (i, 0)))
        args.append(xg)
    in_specs.append(pl.BlockSpec((rows_blk, 3), lambda i: (i, 0)))
    args.append(rel)
    in_specs.append(pl.BlockSpec((cblk, KNBR), lambda i: (i, 0)))
    args.append(mask)
    wlist = [w0, b0, g0, be0, w1, b1, g1, be1, w2, b2]
    for w in wlist:
        in_specs.append(full(w))
        args.append(w)
    c_out = w2.shape[1]
    return pl.pallas_call(
        body,
        grid=(grid,),
        in_specs=in_specs,
        out_specs=pl.BlockSpec((cblk, c_out), lambda i: (i, 0)),
        out_shape=jax.ShapeDtypeStruct((n_centers, c_out), jnp.float32),
    )(*args)


def _mlp2_fused_call(x1, nbr, rel, mask, ws, n_centers, cblk, n_src):
    rows_blk = cblk * KNBR
    grid = n_centers // cblk
    bpc = (n_centers // BB) // cblk
    (w0, b0, g0, be0), (w1, b1, g1, be1), (w2, b2) = ws

    def body(x1_ref, nbr_ref, rel_ref, mask_ref, w0_r, b0_r, g0_r, be0_r,
             w1_r, b1_r, g1_r, be1_r, w2_r, b2_r, out_ref):
        nbr3 = nbr_ref[...][:, :, None]
        jidx = lax.broadcasted_iota(jnp.int32, (cblk, KNBR, n_src), 2)
        onehot = jnp.where(nbr3 == jidx, 1.0, 0.0).reshape(rows_blk, n_src)
        xg = _mm(onehot, x1_ref[...])
        x = jnp.concatenate([xg, rel_ref[...]], axis=1)
        h = _mm(x, w0_r[...]) + b0_r[...]
        h = jnp.maximum(g0_r[...] * (h * _BN_SC) + be0_r[...], 0.0)
        h = _mm(h, w1_r[...]) + b1_r[...]
        h = jnp.maximum(g1_r[...] * (h * _BN_SC) + be1_r[...], 0.0)
        h = _mm(h, w2_r[...]) + b2_r[...]
        c_out = h.shape[-1]
        h = h.reshape(cblk, KNBR, c_out) + mask_ref[...][:, :, None]
        out_ref[...] = jnp.max(h, axis=1)

    full = lambda a: pl.BlockSpec(a.shape, lambda i: (0,) * a.ndim)
    in_specs = [
        pl.BlockSpec((n_src, x1.shape[1]), lambda i: (i // bpc, 0)),
        pl.BlockSpec((cblk, KNBR), lambda i: (i, 0)),
        pl.BlockSpec((rows_blk, 3), lambda i: (i, 0)),
        pl.BlockSpec((cblk, KNBR), lambda i: (i, 0)),
    ]
    args = [x1, nbr, rel, mask]
    for w in [w0, b0, g0, be0, w1, b1, g1, be1, w2, b2]:
        in_specs.append(full(w))
        args.append(w)
    c_out = w2.shape[1]
    return pl.pallas_call(
        body,
        grid=(grid,),
        in_specs=in_specs,
        out_specs=pl.BlockSpec((cblk, c_out), lambda i: (i, 0)),
        out_shape=jax.ShapeDtypeStruct((n_centers, c_out), jnp.float32),
    )(*args)


def _head_call(x2, p2, w3s, wpi, wval):
    npts = x2.shape[0] // BB
    (w0, b0, g0, be0), (w1, b1, g1, be1), (w2, b2) = w3s
    (p0, pb0), (p1, pb1), (p2w, pb2) = wpi
    (v0, vb0), (v1, vb1), (v2, vb2) = wval

    def body(x2_ref, p2_ref, w0_r, b0_r, g0_r, be0_r, w1_r, b1_r, g1_r,
             be1_r, w2_r, b2_r,
             p0_r, pb0_r, p1_r, pb1_r, p2_r, pb2_r,
             v0_r, vb0_r, v1_r, vb1_r, v2_r, vb2_r,
             probs_ref, val_ref):
        x = jnp.concatenate([x2_ref[...], p2_ref[...]], axis=1)
        h = _mm(x, w0_r[...]) + b0_r[...]
        h = jnp.maximum(g0_r[...] * (h * _BN_SC) + be0_r[...], 0.0)
        h = _mm(h, w1_r[...]) + b1_r[...]
        h = jnp.maximum(g1_r[...] * (h * _BN_SC) + be1_r[...], 0.0)
        h = _mm(h, w2_r[...]) + b2_r[...]
        feats = jnp.max(h.reshape(BB, npts, h.shape[-1]), axis=1)
        g = _mm(feats, p0_r[...]) + pb0_r[...]
        g = _mm(g, p1_r[...]) + pb1_r[...]
        logits = _mm(g, p2_r[...]) + pb2_r[...]
        mlog = jnp.max(logits, axis=1, keepdims=True)
        e = jnp.exp(logits - mlog)
        probs_ref[...] = e / jnp.sum(e, axis=1, keepdims=True)
        v = _mm(feats, v0_r[...]) + vb0_r[...]
        v = _mm(v, v1_r[...]) + vb1_r[...]
        val_ref[...] = _mm(v, v2_r[...]) + vb2_r[...]

    args = [x2, p2, w0, b0, g0, be0, w1, b1, g1, be1, w2, b2,
            p0, pb0, p1, pb1, p2w, pb2, v0, vb0, v1, vb1, v2, vb2]
    return pl.pallas_call(
        body,
        out_shape=(jax.ShapeDtypeStruct((BB, 10), jnp.float32),
                   jax.ShapeDtypeStruct((BB, 1), jnp.float32)),
    )(*args)


def _bn_params(p):
    out = []
    for i in range(3):
        w, b = p["Ws"][i], p["bs"][i]
        if i < 2:
            out.append((w, b[None, :], p["gammas"][i][None, :],
                        p["betas"][i][None, :]))
        else:
            out.append((w, b[None, :]))
    return out


@jax.jit
def kernel(pos, batch, params):
    del batch
    pos_t = pos.reshape(BB, NPTS, 3).transpose(0, 2, 1)

    c1 = _fps_call(pos_t, NPTS, NPTS // 2)
    c1_t = c1.transpose(1, 2, 0)
    rel1, mask1 = _group_call(pos_t, c1_t, NPTS, 512, 0.2, False)

    c2 = _fps_call(c1_t, 512, 128)
    c2_t = c2.transpose(1, 2, 0)
    rel2, mask2, nbr2 = _group_call(c1_t, c2_t, 512, 128, 0.4, True)

    x1 = _mlp_pool_call(
        None, rel1.reshape(BB * 512 * KNBR, 3),
        mask1.reshape(BB * 512, KNBR),
        _bn_params(params["sa1"]), BB * 512, 128)

    x2 = _mlp2_fused_call(
        x1, nbr2.reshape(BB * 128, KNBR),
        rel2.reshape(BB * 128 * KNBR, 3),
        mask2.reshape(BB * 128, KNBR),
        _bn_params(params["sa2"]), BB * 128, 64, 512)

    ws3 = _bn_params(params["sa3"])
    wpi = [(w, b[None, :]) for w, b in
           zip(params["pi"]["Ws"], params["pi"]["bs"])]
    wval = [(w, b[None, :]) for w, b in
            zip(params["value"]["Ws"], params["value"]["bs"])]
    p2f = c2.transpose(1, 0, 2).reshape(BB * 128, 3)
    probs, value = _head_call(x2, p2f, ws3, wpi, wval)
    return probs, value[:, 0]


# speedup vs baseline: 14.1701x; 1.0065x over previous
import functools

import numpy as np
import jax
import jax.numpy as jnp
from jax import lax
from jax.experimental import pallas as pl
from jax.experimental.pallas import tpu as pltpu, tpu_sc as plsc

BB = 16
NPTS = 1024
KNBR = 64
NEG = -1e30
_HI = jax.lax.Precision.DEFAULT


def _mm(a, b):
    return jnp.dot(a, b, precision=_HI)
_BN_SC = 1.0 / np.sqrt(1.0 + 1e-5)


def _fps_call(pos_t, n, n_sel):
    def body(pos_ref, out_ref):
        px = pos_ref[:, 0, :]
        py = pos_ref[:, 1, :]
        pz = pos_ref[:, 2, :]
        iota = lax.broadcasted_iota(jnp.int32, (BB, n), 1)

        def step(s, carry):
            dists, last = carry
            onehot = iota == last
            lx = jnp.sum(jnp.where(onehot, px, 0.0), axis=1, keepdims=True)
            ly = jnp.sum(jnp.where(onehot, py, 0.0), axis=1, keepdims=True)
            lz = jnp.sum(jnp.where(onehot, pz, 0.0), axis=1, keepdims=True)
            out_ref[s] = jnp.concatenate([lx, ly, lz], axis=1)
            dx = px - lx
            dy = py - ly
            dz = pz - lz
            d = dx * dx + dy * dy + dz * dz
            dists = jnp.minimum(dists, d)
            m = jnp.max(dists, axis=1, keepdims=True)
            nxt = jnp.min(jnp.where(dists == m, iota, n), axis=1, keepdims=True)
            return dists, nxt

        lax.fori_loop(
            0, n_sel, step,
            (jnp.full((BB, n), 1e30, jnp.float32), jnp.zeros((BB, 1), jnp.int32)),
        )

    return pl.pallas_call(
        body,
        out_shape=jax.ShapeDtypeStruct((n_sel, BB, 3), jnp.float32),
    )(pos_t)


def _group_call(pos_t, cen_t, n, m, radius, emit_nbr):
    info = plsc.get_sparse_core_info()
    nw = info.num_cores * info.num_subcores
    cpw = nw // BB
    mc = m // cpw
    ngrp = n // 16
    r2 = radius * radius

    out_type = [
        jax.ShapeDtypeStruct((BB, m * KNBR * 3), jnp.float32),
        jax.ShapeDtypeStruct((BB, m * KNBR), jnp.float32),
    ]
    scratch = [
        pltpu.VMEM((n,), jnp.float32), pltpu.VMEM((n,), jnp.float32),
        pltpu.VMEM((n,), jnp.float32),
        pltpu.VMEM((mc,), jnp.float32), pltpu.VMEM((mc,), jnp.float32),
        pltpu.VMEM((mc,), jnp.float32),
        pltpu.VMEM((mc * KNBR * 3,), jnp.float32),
        pltpu.VMEM((mc * KNBR,), jnp.float32),
    ]
    if emit_nbr:
        out_type.append(jax.ShapeDtypeStruct((BB, m * KNBR), jnp.int32))
        scratch.append(pltpu.VMEM((mc * KNBR,), jnp.int32))

    mesh = plsc.VectorSubcoreMesh(core_axis_name="c", subcore_axis_name="s")

    def body(*refs):
        if emit_nbr:
            (pxh, pyh, pzh, cxh, cyh, czh, relh, maskh, nbrh,
             px, py, pz, cx, cy, cz, relv, maskv, nbrv) = refs
        else:
            (pxh, pyh, pzh, cxh, cyh, czh, relh, maskh,
             px, py, pz, cx, cy, cz, relv, maskv) = refs
            nbrv = None
        wid = lax.axis_index("s") * info.num_cores + lax.axis_index("c")
        cloud = wid // cpw
        chunk = wid % cpw
        cstart = chunk * mc

        pltpu.sync_copy(pxh.at[cloud], px)
        pltpu.sync_copy(pyh.at[cloud], py)
        pltpu.sync_copy(pzh.at[cloud], pz)
        pltpu.sync_copy(cxh.at[cloud, pl.ds(cstart, mc)], cx)
        pltpu.sync_copy(cyh.at[cloud, pl.ds(cstart, mc)], cy)
        pltpu.sync_copy(czh.at[cloud, pl.ds(cstart, mc)], cz)

        zero16 = jnp.zeros((16,), jnp.float32)
        neg16 = jnp.full((16,), NEG, jnp.float32)
        izero16 = jnp.zeros((16,), jnp.int32)

        def fill(k, _):
            maskv[pl.ds(k * 16, 16)] = neg16
            if emit_nbr:
                nbrv[pl.ds(k * 16, 16)] = izero16
            return 0

        lax.fori_loop(0, mc * KNBR // 16, fill, 0)

        def fillr(k, _):
            relv[pl.ds(k * 16, 16)] = zero16
            return 0

        lax.fori_loop(0, mc * KNBR * 3 // 16, fillr, 0)

        lane = lax.iota(jnp.int32, 16)
        U = 4

        def per_block(blk, _):
            ccx = cx[pl.ds(blk * 16, 16)]
            ccy = cy[pl.ds(blk * 16, 16)]
            ccz = cz[pl.ds(blk * 16, 16)]
            base = (blk * 16 + lane) * KNBR

            def per_pt(jj, cnt):
                for u in range(U):
                    j = jj * U + u
                    jv = jnp.full((16,), j, jnp.int32)
                    vx = plsc.load_gather(px, [jv])
                    vy = plsc.load_gather(py, [jv])
                    vz = plsc.load_gather(pz, [jv])
                    dx = vx - ccx
                    dy = vy - ccy
                    dz = vz - ccz
                    d2 = dx * dx + dy * dy + dz * dz
                    okw = (d2 <= r2) & (cnt < KNBR)
                    slot = base + cnt
                    s3 = slot * 3
                    plsc.store_scatter(relv, [s3], dx, mask=okw)
                    plsc.store_scatter(relv, [s3 + 1], dy, mask=okw)
                    plsc.store_scatter(relv, [s3 + 2], dz, mask=okw)
                    plsc.store_scatter(maskv, [slot], zero16, mask=okw)
                    if emit_nbr:
                        plsc.store_scatter(nbrv, [slot], jv, mask=okw)
                    cnt = cnt + jnp.where(okw, 1, 0)
                return cnt

            lax.fori_loop(0, n // U, per_pt, jnp.zeros((16,), jnp.int32))
            return 0

        lax.fori_loop(0, mc // 16, per_block, 0)

        obase = chunk * (mc * KNBR)
        pltpu.sync_copy(relv, relh.at[cloud, pl.ds(obase * 3, mc * KNBR * 3)])
        pltpu.sync_copy(maskv, maskh.at[cloud, pl.ds(obase, mc * KNBR)])
        if emit_nbr:
            pltpu.sync_copy(nbrv, nbrh.at[cloud, pl.ds(obase, mc * KNBR)])

    fn = pl.kernel(body, out_type=tuple(out_type), mesh=mesh,
                   scratch_types=scratch,
                   compiler_params=pltpu.CompilerParams(
                       needs_layout_passes=False))
    return fn(pos_t[:, 0, :], pos_t[:, 1, :], pos_t[:, 2, :],
              cen_t[:, 0, :], cen_t[:, 1, :], cen_t[:, 2, :])


def _gather_rows(table, idx):
    q, d = idx.shape[0], table.shape[1]
    info = plsc.get_sparse_core_info()
    nw = info.num_cores * info.num_subcores
    per_w = q // nw
    chunk = 128
    nch = per_w // chunk
    mesh = plsc.VectorSubcoreMesh(core_axis_name="c", subcore_axis_name="s")

    nb = 3

    def body(tab_h, idx_h, out_h, idx_v, rows_v, sem_g, sem_o):
        wid = lax.axis_index("s") * info.num_cores + lax.axis_index("c")
        base = wid * per_w
        pltpu.sync_copy(idx_h.at[pl.ds(base, per_w)], idx_v)

        def gat(t):
            return pltpu.async_copy(
                tab_h.at[idx_v.at[pl.ds(t * chunk, chunk)]],
                rows_v.at[t % nb], sem_g.at[t % nb])

        def put(t):
            return pltpu.async_copy(
                rows_v.at[t % nb],
                out_h.at[pl.ds(base + t * chunk, chunk)], sem_o.at[t % nb])

        gh = {0: gat(0), 1: gat(1)}
        wh = {}
        for t in range(nch):
            gh[t].wait()
            wh[t] = put(t)
            nt = t + 2
            if nt < nch:
                if nt >= nb:
                    wh[nt - nb].wait()
                gh[nt] = gat(nt)
        wh[nch - 3].wait()
        wh[nch - 2].wait()
        wh[nch - 1].wait()

    fn = pl.kernel(
        body,
        out_type=jax.ShapeDtypeStruct((q, d), jnp.float32),
        mesh=mesh,
        scratch_types=[
            pltpu.VMEM((per_w,), jnp.int32),
            pltpu.VMEM((nb, chunk, d), jnp.float32),
            pltpu.SemaphoreType.DMA((nb,)),
            pltpu.SemaphoreType.DMA((nb,)),
        ],
        compiler_params=pltpu.CompilerParams(needs_layout_passes=False),
    )
    return fn(table, idx)


def _mlp_pool_call(xg, rel, mask, ws, n_centers, cblk):
    rows_blk = cblk * KNBR
    grid = n_centers // cblk
    (w0, b0, g0, be0), (w1, b1, g1, be1), (w2, b2) = ws

    def body(*refs):
        if xg is None:
            (rel_ref, mask_ref, w0_r, b0_r, g0_r, be0_r, w1_r, b1_r, g1_r,
             be1_r, w2_r, b2_r, out_ref) = refs
            x = rel_ref[...]
        else:
            (xg_ref, rel_ref, mask_ref, w0_r, b0_r, g0_r, be0_r, w1_r, b1_r,
             g1_r, be1_r, w2_r, b2_r, out_ref) = refs
            x = jnp.concatenate([xg_ref[...], rel_ref[...]], axis=1)
        h = _mm(x, w0_r[...]) + b0_r[...]
        h = jnp.maximum(g0_r[...] * (h * _BN_SC) + be0_r[...], 0.0)
        h = _mm(h, w1_r[...]) + b1_r[...]
        h = jnp.maximum(g1_r[...] * (h * _BN_SC) + be1_r[...], 0.0)
        h = _mm(h, w2_r[...]) + b2_r[...]
        c_out = h.shape[-1]
        h = h.reshape(cblk, KNBR, c_out) + mask_ref[...][:, :, None]
        out_ref[...] = jnp.max(h, axis=1)

    full = lambda a: pl.BlockSpec(a.shape, lambda i: (0,) * a.ndim)
    in_specs = []
    args = []
    if xg is not None:
        in_specs.append(pl.BlockSpec((rows_blk, xg.shape[1]), lambda i: (i, 0)))
        args.append(xg)
    in_specs.append(pl.BlockSpec((rows_blk, 3), lambda i: (i, 0)))
    args.append(rel)
    in_specs.append(pl.BlockSpec((cblk, KNBR), lambda i: (i, 0)))
    args.append(mask)
    wlist = [w0, b0, g0, be0, w1, b1, g1, be1, w2, b2]
    for w in wlist:
        in_specs.append(full(w))
        args.append(w)
    c_out = w2.shape[1]
    return pl.pallas_call(
        body,
        grid=(grid,),
        in_specs=in_specs,
        out_specs=pl.BlockSpec((cblk, c_out), lambda i: (i, 0)),
        out_shape=jax.ShapeDtypeStruct((n_centers, c_out), jnp.float32),
    )(*args)


def _mlp2_fused_call(x1, nbr, rel, mask, ws, n_centers, cblk, n_src):
    rows_blk = cblk * KNBR
    grid = n_centers // cblk
    bpc = (n_centers // BB) // cblk
    (w0, b0, g0, be0), (w1, b1, g1, be1), (w2, b2) = ws

    def body(x1_ref, nbr_ref, rel_ref, mask_ref, w0_r, b0_r, g0_r, be0_r,
             w1_r, b1_r, g1_r, be1_r, w2_r, b2_r, out_ref):
        nbr3 = nbr_ref[...][:, :, None]
        jidx = lax.broadcasted_iota(jnp.int32, (cblk, KNBR, n_src), 2)
        onehot = jnp.where(nbr3 == jidx, 1.0, 0.0).reshape(rows_blk, n_src)
        xg = _mm(onehot, x1_ref[...])
        x = jnp.concatenate([xg, rel_ref[...]], axis=1)
        h = _mm(x, w0_r[...]) + b0_r[...]
        h = jnp.maximum(g0_r[...] * (h * _BN_SC) + be0_r[...], 0.0)
        h = _mm(h, w1_r[...]) + b1_r[...]
        h = jnp.maximum(g1_r[...] * (h * _BN_SC) + be1_r[...], 0.0)
        h = _mm(h, w2_r[...]) + b2_r[...]
        c_out = h.shape[-1]
        h = h.reshape(cblk, KNBR, c_out) + mask_ref[...][:, :, None]
        out_ref[...] = jnp.max(h, axis=1)

    full = lambda a: pl.BlockSpec(a.shape, lambda i: (0,) * a.ndim)
    in_specs = [
        pl.BlockSpec((n_src, x1.shape[1]), lambda i: (i // bpc, 0)),
        pl.BlockSpec((cblk, KNBR), lambda i: (i, 0)),
        pl.BlockSpec((rows_blk, 3), lambda i: (i, 0)),
        pl.BlockSpec((cblk, KNBR), lambda i: (i, 0)),
    ]
    args = [x1, nbr, rel, mask]
    for w in [w0, b0, g0, be0, w1, b1, g1, be1, w2, b2]:
        in_specs.append(full(w))
        args.append(w)
    c_out = w2.shape[1]
    return pl.pallas_call(
        body,
        grid=(grid,),
        in_specs=in_specs,
        out_specs=pl.BlockSpec((cblk, c_out), lambda i: (i, 0)),
        out_shape=jax.ShapeDtypeStruct((n_centers, c_out), jnp.float32),
    )(*args)


def _head_call(x2, p2, w3s, wpi, wval):
    npts = x2.shape[0] // BB
    (w0, b0, g0, be0), (w1, b1, g1, be1), (w2, b2) = w3s
    (p0, pb0), (p1, pb1), (p2w, pb2) = wpi
    (v0, vb0), (v1, vb1), (v2, vb2) = wval

    def body(x2_ref, p2_ref, w0_r, b0_r, g0_r, be0_r, w1_r, b1_r, g1_r,
             be1_r, w2_r, b2_r,
             p0_r, pb0_r, p1_r, pb1_r, p2_r, pb2_r,
             v0_r, vb0_r, v1_r, vb1_r, v2_r, vb2_r,
             probs_ref, val_ref):
        x = jnp.concatenate([x2_ref[...], p2_ref[...]], axis=1)
        h = _mm(x, w0_r[...]) + b0_r[...]
        h = jnp.maximum(g0_r[...] * (h * _BN_SC) + be0_r[...], 0.0)
        h = _mm(h, w1_r[...]) + b1_r[...]
        h = jnp.maximum(g1_r[...] * (h * _BN_SC) + be1_r[...], 0.0)
        h = _mm(h, w2_r[...]) + b2_r[...]
        feats = jnp.max(h.reshape(BB, npts, h.shape[-1]), axis=1)
        g = _mm(feats, p0_r[...]) + pb0_r[...]
        g = _mm(g, p1_r[...]) + pb1_r[...]
        logits = _mm(g, p2_r[...]) + pb2_r[...]
        mlog = jnp.max(logits, axis=1, keepdims=True)
        e = jnp.exp(logits - mlog)
        probs_ref[...] = e / jnp.sum(e, axis=1, keepdims=True)
        v = _mm(feats, v0_r[...]) + vb0_r[...]
        v = _mm(v, v1_r[...]) + vb1_r[...]
        val_ref[...] = _mm(v, v2_r[...]) + vb2_r[...]

    args = [x2, p2, w0, b0, g0, be0, w1, b1, g1, be1, w2, b2,
            p0, pb0, p1, pb1, p2w, pb2, v0, vb0, v1, vb1, v2, vb2]
    return pl.pallas_call(
        body,
        out_shape=(jax.ShapeDtypeStruct((BB, 10), jnp.float32),
                   jax.ShapeDtypeStruct((BB, 1), jnp.float32)),
    )(*args)


def _bn_params(p):
    out = []
    for i in range(3):
        w, b = p["Ws"][i], p["bs"][i]
        if i < 2:
            out.append((w, b[None, :], p["gammas"][i][None, :],
                        p["betas"][i][None, :]))
        else:
            out.append((w, b[None, :]))
    return out


@jax.jit
def kernel(pos, batch, params):
    del batch
    pos_t = pos.reshape(BB, NPTS, 3).transpose(0, 2, 1)

    c1 = _fps_call(pos_t, NPTS, NPTS // 2)
    c1_t = c1.transpose(1, 2, 0)
    rel1, mask1 = _group_call(pos_t, c1_t, NPTS, 512, 0.2, False)

    c2 = _fps_call(c1_t, 512, 128)
    c2_t = c2.transpose(1, 2, 0)
    rel2, mask2, nbr2 = _group_call(c1_t, c2_t, 512, 128, 0.4, True)

    x1 = _mlp_pool_call(
        None, rel1.reshape(BB * 512 * KNBR, 3),
        mask1.reshape(BB * 512, KNBR),
        _bn_params(params["sa1"]), BB * 512, 256)

    x2 = _mlp2_fused_call(
        x1, nbr2.reshape(BB * 128, KNBR),
        rel2.reshape(BB * 128 * KNBR, 3),
        mask2.reshape(BB * 128, KNBR),
        _bn_params(params["sa2"]), BB * 128, 64, 512)

    ws3 = _bn_params(params["sa3"])
    wpi = [(w, b[None, :]) for w, b in
           zip(params["pi"]["Ws"], params["pi"]["bs"])]
    wval = [(w, b[None, :]) for w, b in
            zip(params["value"]["Ws"], params["value"]["bs"])]
    p2f = c2.transpose(1, 0, 2).reshape(BB * 128, 3)
    probs, value = _head_call(x2, p2f, ws3, wpi, wval)
    return probs, value[:, 0]


# speedup vs baseline: 14.1767x; 1.0005x over previous
import numpy as np
import jax
import jax.numpy as jnp
from jax import lax
from jax.experimental import pallas as pl
from jax.experimental.pallas import tpu as pltpu, tpu_sc as plsc

BB = 16
NPTS = 1024
KNBR = 64
NEG = -1e30
_HI = jax.lax.Precision.DEFAULT


def _mm(a, b):
    return jnp.dot(a, b, precision=_HI)
_BN_SC = 1.0 / np.sqrt(1.0 + 1e-5)


def _fps_call(pos_t, n, n_sel):
    def body(pos_ref, out_ref):
        px = pos_ref[:, 0, :]
        py = pos_ref[:, 1, :]
        pz = pos_ref[:, 2, :]
        iota = lax.broadcasted_iota(jnp.int32, (BB, n), 1)

        def step(s, carry):
            dists, last = carry
            onehot = iota == last
            lx = jnp.sum(jnp.where(onehot, px, 0.0), axis=1, keepdims=True)
            ly = jnp.sum(jnp.where(onehot, py, 0.0), axis=1, keepdims=True)
            lz = jnp.sum(jnp.where(onehot, pz, 0.0), axis=1, keepdims=True)
            out_ref[s] = jnp.concatenate([lx, ly, lz], axis=1)
            dx = px - lx
            dy = py - ly
            dz = pz - lz
            d = dx * dx + dy * dy + dz * dz
            dists = jnp.minimum(dists, d)
            m = jnp.max(dists, axis=1, keepdims=True)
            nxt = jnp.min(jnp.where(dists == m, iota, n), axis=1, keepdims=True)
            return dists, nxt

        lax.fori_loop(
            0, n_sel, step,
            (jnp.full((BB, n), 1e30, jnp.float32), jnp.zeros((BB, 1), jnp.int32)),
        )

    return pl.pallas_call(
        body,
        out_shape=jax.ShapeDtypeStruct((n_sel, BB, 3), jnp.float32),
    )(pos_t)


def _group_call(pos_t, cen_t, n, m, radius, emit_nbr):
    info = plsc.get_sparse_core_info()
    nw = info.num_cores * info.num_subcores
    cpw = nw // BB
    mc = m // cpw
    ngrp = n // 16
    r2 = radius * radius

    out_type = [
        jax.ShapeDtypeStruct((BB, m * KNBR * 3), jnp.float32),
        jax.ShapeDtypeStruct((BB, m * KNBR), jnp.float32),
    ]
    scratch = [
        pltpu.VMEM((n,), jnp.float32), pltpu.VMEM((n,), jnp.float32),
        pltpu.VMEM((n,), jnp.float32),
        pltpu.VMEM((mc,), jnp.float32), pltpu.VMEM((mc,), jnp.float32),
        pltpu.VMEM((mc,), jnp.float32),
        pltpu.VMEM((mc * KNBR * 3,), jnp.float32),
        pltpu.VMEM((mc * KNBR,), jnp.float32),
    ]
    if emit_nbr:
        out_type.append(jax.ShapeDtypeStruct((BB, m * KNBR), jnp.int32))
        scratch.append(pltpu.VMEM((mc * KNBR,), jnp.int32))

    mesh = plsc.VectorSubcoreMesh(core_axis_name="c", subcore_axis_name="s")

    def body(*refs):
        if emit_nbr:
            (pxh, pyh, pzh, cxh, cyh, czh, relh, maskh, nbrh,
             px, py, pz, cx, cy, cz, relv, maskv, nbrv) = refs
        else:
            (pxh, pyh, pzh, cxh, cyh, czh, relh, maskh,
             px, py, pz, cx, cy, cz, relv, maskv) = refs
            nbrv = None
        wid = lax.axis_index("s") * info.num_cores + lax.axis_index("c")
        cloud = wid // cpw
        chunk = wid % cpw
        cstart = chunk * mc

        pltpu.sync_copy(pxh.at[cloud], px)
        pltpu.sync_copy(pyh.at[cloud], py)
        pltpu.sync_copy(pzh.at[cloud], pz)
        pltpu.sync_copy(cxh.at[cloud, pl.ds(cstart, mc)], cx)
        pltpu.sync_copy(cyh.at[cloud, pl.ds(cstart, mc)], cy)
        pltpu.sync_copy(czh.at[cloud, pl.ds(cstart, mc)], cz)

        zero16 = jnp.zeros((16,), jnp.float32)
        neg16 = jnp.full((16,), NEG, jnp.float32)
        izero16 = jnp.zeros((16,), jnp.int32)

        def fill(k, _):
            maskv[pl.ds(k * 16, 16)] = neg16
            if emit_nbr:
                nbrv[pl.ds(k * 16, 16)] = izero16
            return 0

        lax.fori_loop(0, mc * KNBR // 16, fill, 0)

        def fillr(k, _):
            relv[pl.ds(k * 16, 16)] = zero16
            return 0

        lax.fori_loop(0, mc * KNBR * 3 // 16, fillr, 0)

        lane = lax.iota(jnp.int32, 16)
        U = 4

        def per_block(blk, _):
            ccx = cx[pl.ds(blk * 16, 16)]
            ccy = cy[pl.ds(blk * 16, 16)]
            ccz = cz[pl.ds(blk * 16, 16)]
            base = (blk * 16 + lane) * KNBR

            def per_pt(jj, cnt):
                for u in range(U):
                    j = jj * U + u
                    jv = jnp.full((16,), j, jnp.int32)
                    vx = plsc.load_gather(px, [jv])
                    vy = plsc.load_gather(py, [jv])
                    vz = plsc.load_gather(pz, [jv])
                    dx = vx - ccx
                    dy = vy - ccy
                    dz = vz - ccz
                    d2 = dx * dx + dy * dy + dz * dz
                    okw = (d2 <= r2) & (cnt < KNBR)
                    slot = base + cnt
                    s3 = slot * 3
                    plsc.store_scatter(relv, [s3], dx, mask=okw)
                    plsc.store_scatter(relv, [s3 + 1], dy, mask=okw)
                    plsc.store_scatter(relv, [s3 + 2], dz, mask=okw)
                    plsc.store_scatter(maskv, [slot], zero16, mask=okw)
                    if emit_nbr:
                        plsc.store_scatter(nbrv, [slot], jv, mask=okw)
                    cnt = cnt + jnp.where(okw, 1, 0)
                return cnt

            lax.fori_loop(0, n // U, per_pt, jnp.zeros((16,), jnp.int32))
            return 0

        lax.fori_loop(0, mc // 16, per_block, 0)

        obase = chunk * (mc * KNBR)
        pltpu.sync_copy(relv, relh.at[cloud, pl.ds(obase * 3, mc * KNBR * 3)])
        pltpu.sync_copy(maskv, maskh.at[cloud, pl.ds(obase, mc * KNBR)])
        if emit_nbr:
            pltpu.sync_copy(nbrv, nbrh.at[cloud, pl.ds(obase, mc * KNBR)])

    fn = pl.kernel(body, out_type=tuple(out_type), mesh=mesh,
                   scratch_types=scratch,
                   compiler_params=pltpu.CompilerParams(
                       needs_layout_passes=False))
    return fn(pos_t[:, 0, :], pos_t[:, 1, :], pos_t[:, 2, :],
              cen_t[:, 0, :], cen_t[:, 1, :], cen_t[:, 2, :])


def _mlp_pool_call(xg, rel, mask, ws, n_centers, cblk):
    rows_blk = cblk * KNBR
    grid = n_centers // cblk
    (w0, b0, g0, be0), (w1, b1, g1, be1), (w2, b2) = ws

    def body(*refs):
        if xg is None:
            (rel_ref, mask_ref, w0_r, b0_r, g0_r, be0_r, w1_r, b1_r, g1_r,
             be1_r, w2_r, b2_r, out_ref) = refs
            x = rel_ref[...]
        else:
            (xg_ref, rel_ref, mask_ref, w0_r, b0_r, g0_r, be0_r, w1_r, b1_r,
             g1_r, be1_r, w2_r, b2_r, out_ref) = refs
            x = jnp.concatenate([xg_ref[...], rel_ref[...]], axis=1)
        h = _mm(x, w0_r[...]) + b0_r[...]
        h = jnp.maximum(g0_r[...] * (h * _BN_SC) + be0_r[...], 0.0)
        h = _mm(h, w1_r[...]) + b1_r[...]
        h = jnp.maximum(g1_r[...] * (h * _BN_SC) + be1_r[...], 0.0)
        h = _mm(h, w2_r[...]) + b2_r[...]
        c_out = h.shape[-1]
        h = h.reshape(cblk, KNBR, c_out) + mask_ref[...][:, :, None]
        out_ref[...] = jnp.max(h, axis=1)

    full = lambda a: pl.BlockSpec(a.shape, lambda i: (0,) * a.ndim)
    in_specs = []
    args = []
    if xg is not None:
        in_specs.append(pl.BlockSpec((rows_blk, xg.shape[1]), lambda i: (i, 0)))
        args.append(xg)
    in_specs.append(pl.BlockSpec((rows_blk, 3), lambda i: (i, 0)))
    args.append(rel)
    in_specs.append(pl.BlockSpec((cblk, KNBR), lambda i: (i, 0)))
    args.append(mask)
    wlist = [w0, b0, g0, be0, w1, b1, g1, be1, w2, b2]
    for w in wlist:
        in_specs.append(full(w))
        args.append(w)
    c_out = w2.shape[1]
    return pl.pallas_call(
        body,
        grid=(grid,),
        in_specs=in_specs,
        out_specs=pl.BlockSpec((cblk, c_out), lambda i: (i, 0)),
        out_shape=jax.ShapeDtypeStruct((n_centers, c_out), jnp.float32),
    )(*args)


def _mlp2_fused_call(x1, nbr, rel, mask, ws, n_centers, cblk, n_src):
    rows_blk = cblk * KNBR
    grid = n_centers // cblk
    bpc = (n_centers // BB) // cblk
    (w0, b0, g0, be0), (w1, b1, g1, be1), (w2, b2) = ws

    def body(x1_ref, nbr_ref, rel_ref, mask_ref, w0_r, b0_r, g0_r, be0_r,
             w1_r, b1_r, g1_r, be1_r, w2_r, b2_r, out_ref):
        nbr3 = nbr_ref[...][:, :, None]
        jidx = lax.broadcasted_iota(jnp.int32, (cblk, KNBR, n_src), 2)
        onehot = jnp.where(nbr3 == jidx, 1.0, 0.0).reshape(rows_blk, n_src)
        xg = _mm(onehot, x1_ref[...])
        x = jnp.concatenate([xg, rel_ref[...]], axis=1)
        h = _mm(x, w0_r[...]) + b0_r[...]
        h = jnp.maximum(g0_r[...] * (h * _BN_SC) + be0_r[...], 0.0)
        h = _mm(h, w1_r[...]) + b1_r[...]
        h = jnp.maximum(g1_r[...] * (h * _BN_SC) + be1_r[...], 0.0)
        h = _mm(h, w2_r[...]) + b2_r[...]
        c_out = h.shape[-1]
        h = h.reshape(cblk, KNBR, c_out) + mask_ref[...][:, :, None]
        out_ref[...] = jnp.max(h, axis=1)

    full = lambda a: pl.BlockSpec(a.shape, lambda i: (0,) * a.ndim)
    in_specs = [
        pl.BlockSpec((n_src, x1.shape[1]), lambda i: (i // bpc, 0)),
        pl.BlockSpec((cblk, KNBR), lambda i: (i, 0)),
        pl.BlockSpec((rows_blk, 3), lambda i: (i, 0)),
        pl.BlockSpec((cblk, KNBR), lambda i: (i, 0)),
    ]
    args = [x1, nbr, rel, mask]
    for w in [w0, b0, g0, be0, w1, b1, g1, be1, w2, b2]:
        in_specs.append(full(w))
        args.append(w)
    c_out = w2.shape[1]
    return pl.pallas_call(
        body,
        grid=(grid,),
        in_specs=in_specs,
        out_specs=pl.BlockSpec((cblk, c_out), lambda i: (i, 0)),
        out_shape=jax.ShapeDtypeStruct((n_centers, c_out), jnp.float32),
    )(*args)


def _head_call(x2, p2, w3s, wpi, wval):
    npts = x2.shape[0] // BB
    (w0, b0, g0, be0), (w1, b1, g1, be1), (w2, b2) = w3s
    (p0, pb0), (p1, pb1), (p2w, pb2) = wpi
    (v0, vb0), (v1, vb1), (v2, vb2) = wval

    def body(x2_ref, p2_ref, w0_r, b0_r, g0_r, be0_r, w1_r, b1_r, g1_r,
             be1_r, w2_r, b2_r,
             p0_r, pb0_r, p1_r, pb1_r, p2_r, pb2_r,
             v0_r, vb0_r, v1_r, vb1_r, v2_r, vb2_r,
             probs_ref, val_ref):
        x = jnp.concatenate([x2_ref[...], p2_ref[...]], axis=1)
        h = _mm(x, w0_r[...]) + b0_r[...]
        h = jnp.maximum(g0_r[...] * (h * _BN_SC) + be0_r[...], 0.0)
        h = _mm(h, w1_r[...]) + b1_r[...]
        h = jnp.maximum(g1_r[...] * (h * _BN_SC) + be1_r[...], 0.0)
        h = _mm(h, w2_r[...]) + b2_r[...]
        feats = jnp.max(h.reshape(BB, npts, h.shape[-1]), axis=1)
        g = _mm(feats, p0_r[...]) + pb0_r[...]
        g = _mm(g, p1_r[...]) + pb1_r[...]
        logits = _mm(g, p2_r[...]) + pb2_r[...]
        mlog = jnp.max(logits, axis=1, keepdims=True)
        e = jnp.exp(logits - mlog)
        probs_ref[...] = e / jnp.sum(e, axis=1, keepdims=True)
        v = _mm(feats, v0_r[...]) + vb0_r[...]
        v = _mm(v, v1_r[...]) + vb1_r[...]
        val_ref[...] = _mm(v, v2_r[...]) + vb2_r[...]

    args = [x2, p2, w0, b0, g0, be0, w1, b1, g1, be1, w2, b2,
            p0, pb0, p1, pb1, p2w, pb2, v0, vb0, v1, vb1, v2, vb2]
    return pl.pallas_call(
        body,
        out_shape=(jax.ShapeDtypeStruct((BB, 10), jnp.float32),
                   jax.ShapeDtypeStruct((BB, 1), jnp.float32)),
    )(*args)


def _bn_params(p):
    out = []
    for i in range(3):
        w, b = p["Ws"][i], p["bs"][i]
        if i < 2:
            out.append((w, b[None, :], p["gammas"][i][None, :],
                        p["betas"][i][None, :]))
        else:
            out.append((w, b[None, :]))
    return out


@jax.jit
def kernel(pos, batch, params):
    del batch
    pos_t = pos.reshape(BB, NPTS, 3).transpose(0, 2, 1)

    c1 = _fps_call(pos_t, NPTS, NPTS // 2)
    c1_t = c1.transpose(1, 2, 0)
    rel1, mask1 = _group_call(pos_t, c1_t, NPTS, 512, 0.2, False)

    c2 = _fps_call(c1_t, 512, 128)
    c2_t = c2.transpose(1, 2, 0)
    rel2, mask2, nbr2 = _group_call(c1_t, c2_t, 512, 128, 0.4, True)

    x1 = _mlp_pool_call(
        None, rel1.reshape(BB * 512 * KNBR, 3),
        mask1.reshape(BB * 512, KNBR),
        _bn_params(params["sa1"]), BB * 512, 256)

    x2 = _mlp2_fused_call(
        x1, nbr2.reshape(BB * 128, KNBR),
        rel2.reshape(BB * 128 * KNBR, 3),
        mask2.reshape(BB * 128, KNBR),
        _bn_params(params["sa2"]), BB * 128, 64, 512)

    ws3 = _bn_params(params["sa3"])
    wpi = [(w, b[None, :]) for w, b in
           zip(params["pi"]["Ws"], params["pi"]["bs"])]
    wval = [(w, b[None, :]) for w, b in
            zip(params["value"]["Ws"], params["value"]["bs"])]
    p2f = c2.transpose(1, 0, 2).reshape(BB * 128, 3)
    probs, value = _head_call(x2, p2f, ws3, wpi, wval)
    return probs, value[:, 0]
